# baseline (speedup 1.0000x reference)
import jax
import jax.numpy as jnp
from jax.experimental import pallas as pl
from jax.experimental.pallas import tpu as pltpu

EPS = 1e-07
B, C, H, W = 16, 4, 512, 512


def _dice_kernel(dom_ref, x_ref, t_ref, out_ref, acc_ref):
    i = pl.program_id(0)
    n = pl.num_programs(0)

    @pl.when(i == 0)
    def _init():
        acc_ref[...] = jnp.zeros_like(acc_ref)

    xb = x_ref[0]
    tb = t_ref[0]
    s_i = jnp.sum(xb * tb)
    s_c = jnp.sum(xb) + jnp.sum(tb)

    b = i // C
    c = i % C
    row = jax.lax.broadcasted_iota(jnp.int32, (B, C), 0)
    col = jax.lax.broadcasted_iota(jnp.int32, (B, C), 1)
    onehot = jnp.where((row == b) & (col == c), 1.0, 0.0)
    acc_ref[0] += s_i * onehot
    acc_ref[1] += s_c * onehot

    @pl.when(i == n - 1)
    def _epilogue():
        inter = acc_ref[0]
        card = acc_ref[1]
        d0 = dom_ref[:, 0:1]
        d1 = dom_ref[:, 1:2]
        w1 = (d1 > d0).astype(jnp.float32)
        w0 = 1.0 - w1
        i0 = jnp.sum(inter * w0, axis=0, keepdims=True)
        c0 = jnp.sum(card * w0, axis=0, keepdims=True)
        i1 = jnp.sum(inter * w1, axis=0, keepdims=True)
        c1 = jnp.sum(card * w1, axis=0, keepdims=True)
        loss0 = 1.0 - jnp.mean(2.0 * i0 / (c0 + EPS))
        loss1 = 1.0 - jnp.mean(2.0 * i1 / (c1 + EPS))
        lane = jax.lax.broadcasted_iota(jnp.int32, (1, 4), 1)
        out_ref[...] = jnp.where(
            lane == 0, loss0 + loss1, jnp.where(lane == 1, loss0, loss1)
        )


def kernel(x, label_true, domains):
    xr = x.reshape(B * C, H, W)
    tr = label_true.reshape(B * C, H, W)
    out = pl.pallas_call(
        _dice_kernel,
        grid=(B * C,),
        in_specs=[
            pl.BlockSpec((B, 2), lambda i: (0, 0)),
            pl.BlockSpec((1, H, W), lambda i: (i, 0, 0)),
            pl.BlockSpec((1, H, W), lambda i: (i, 0, 0)),
        ],
        out_specs=pl.BlockSpec((1, 4), lambda i: (0, 0)),
        out_shape=jax.ShapeDtypeStruct((1, 4), jnp.float32),
        scratch_shapes=[pltpu.VMEM((2, B, C), jnp.float32)],
    )(domains, xr, tr)
    return (out[0, 0], (out[0, 1], out[0, 2]))


# speedup vs baseline: 1.3814x; 1.3814x over previous
import jax
import jax.numpy as jnp
from jax.experimental import pallas as pl
from jax.experimental.pallas import tpu as pltpu

EPS = 1e-07
B, C, H, W = 16, 4, 512, 512
HALF = (B * C) // 2


def _dice_kernel(dom_ref, xa_ref, ta_ref, xb_ref, tb_ref, out_ref, acc_ref):
    i = pl.program_id(0)
    n = pl.num_programs(0)

    @pl.when(i == 0)
    def _init():
        acc_ref[...] = jnp.zeros_like(acc_ref)

    xa = xa_ref[0]
    ta = ta_ref[0]
    xb = xb_ref[0]
    tb = tb_ref[0]
    si_a = jnp.sum(xa * ta)
    sc_a = jnp.sum(xa + ta)
    si_b = jnp.sum(xb * tb)
    sc_b = jnp.sum(xb + tb)

    b = i // C
    c = i % C
    row = jax.lax.broadcasted_iota(jnp.int32, (B, C), 0)
    col = jax.lax.broadcasted_iota(jnp.int32, (B, C), 1)
    hot_a = (row == b) & (col == c)
    hot_b = (row == b + HALF // C) & (col == c)
    acc_ref[0] += jnp.where(hot_a, si_a, 0.0) + jnp.where(hot_b, si_b, 0.0)
    acc_ref[1] += jnp.where(hot_a, sc_a, 0.0) + jnp.where(hot_b, sc_b, 0.0)

    @pl.when(i == n - 1)
    def _epilogue():
        inter = acc_ref[0]
        card = acc_ref[1]
        d0 = dom_ref[:, 0:1]
        d1 = dom_ref[:, 1:2]
        w1 = (d1 > d0).astype(jnp.float32)
        w0 = 1.0 - w1
        i0 = jnp.sum(inter * w0, axis=0, keepdims=True)
        c0 = jnp.sum(card * w0, axis=0, keepdims=True)
        i1 = jnp.sum(inter * w1, axis=0, keepdims=True)
        c1 = jnp.sum(card * w1, axis=0, keepdims=True)
        loss0 = 1.0 - jnp.mean(2.0 * i0 / (c0 + EPS))
        loss1 = 1.0 - jnp.mean(2.0 * i1 / (c1 + EPS))
        lane = jax.lax.broadcasted_iota(jnp.int32, (1, 4), 1)
        out_ref[...] = jnp.where(
            lane == 0, loss0 + loss1, jnp.where(lane == 1, loss0, loss1)
        )


def kernel(x, label_true, domains):
    xr = x.reshape(B * C, H, W)
    tr = label_true.reshape(B * C, H, W)
    out = pl.pallas_call(
        _dice_kernel,
        grid=(HALF,),
        in_specs=[
            pl.BlockSpec((B, 2), lambda i: (0, 0)),
            pl.BlockSpec((1, H, W), lambda i: (i, 0, 0)),
            pl.BlockSpec((1, H, W), lambda i: (i, 0, 0)),
            pl.BlockSpec((1, H, W), lambda i: (i + HALF, 0, 0)),
            pl.BlockSpec((1, H, W), lambda i: (i + HALF, 0, 0)),
        ],
        out_specs=pl.BlockSpec((1, 4), lambda i: (0, 0)),
        out_shape=jax.ShapeDtypeStruct((1, 4), jnp.float32),
        scratch_shapes=[pltpu.VMEM((2, B, C), jnp.float32)],
    )(domains, xr, tr, xr, tr)
    return (out[0, 0], (out[0, 1], out[0, 2]))


# speedup vs baseline: 1.5698x; 1.1364x over previous
import jax
import jax.numpy as jnp
from jax.experimental import pallas as pl
from jax.experimental.pallas import tpu as pltpu

EPS = 1e-07
B, C, H, W = 16, 4, 512, 512
HALF = (B * C) // 2
QUARTER = (B * C) // 4


def _dice_kernel(dom_ref, x0_ref, t0_ref, x1_ref, t1_ref, x2_ref, t2_ref,
                 x3_ref, t3_ref, out_ref, acc_ref):
    i = pl.program_id(0)
    n = pl.num_programs(0)

    @pl.when(i == 0)
    def _init():
        acc_ref[...] = jnp.zeros_like(acc_ref)

    row = jax.lax.broadcasted_iota(jnp.int32, (B, C), 0)
    col = jax.lax.broadcasted_iota(jnp.int32, (B, C), 1)
    b = i // C
    c = i % C
    acc_i = jnp.zeros((B, C), jnp.float32)
    acc_c = jnp.zeros((B, C), jnp.float32)
    for q, (xq_ref, tq_ref) in enumerate(
        ((x0_ref, t0_ref), (x1_ref, t1_ref), (x2_ref, t2_ref), (x3_ref, t3_ref))
    ):
        xq = xq_ref[0]
        tq = tq_ref[0]
        hot = (row == b + q * (QUARTER // C)) & (col == c)
        acc_i += jnp.where(hot, jnp.sum(xq * tq), 0.0)
        acc_c += jnp.where(hot, jnp.sum(xq + tq), 0.0)
    acc_ref[0] += acc_i
    acc_ref[1] += acc_c

    @pl.when(i == n - 1)
    def _epilogue():
        inter = acc_ref[0]
        card = acc_ref[1]
        d0 = dom_ref[:, 0:1]
        d1 = dom_ref[:, 1:2]
        w1 = (d1 > d0).astype(jnp.float32)
        w0 = 1.0 - w1
        i0 = jnp.sum(inter * w0, axis=0, keepdims=True)
        c0 = jnp.sum(card * w0, axis=0, keepdims=True)
        i1 = jnp.sum(inter * w1, axis=0, keepdims=True)
        c1 = jnp.sum(card * w1, axis=0, keepdims=True)
        loss0 = 1.0 - jnp.mean(2.0 * i0 / (c0 + EPS))
        loss1 = 1.0 - jnp.mean(2.0 * i1 / (c1 + EPS))
        lane = jax.lax.broadcasted_iota(jnp.int32, (1, 4), 1)
        out_ref[...] = jnp.where(
            lane == 0, loss0 + loss1, jnp.where(lane == 1, loss0, loss1)
        )


def kernel(x, label_true, domains):
    xr = x.reshape(B * C, H, W)
    tr = label_true.reshape(B * C, H, W)
    specs = [pl.BlockSpec((B, 2), lambda i: (0, 0))]
    for q in range(4):
        specs.append(pl.BlockSpec((1, H, W), lambda i, q=q: (i + q * QUARTER, 0, 0)))
        specs.append(pl.BlockSpec((1, H, W), lambda i, q=q: (i + q * QUARTER, 0, 0)))
    out = pl.pallas_call(
        _dice_kernel,
        grid=(QUARTER,),
        in_specs=specs,
        out_specs=pl.BlockSpec((1, 4), lambda i: (0, 0)),
        out_shape=jax.ShapeDtypeStruct((1, 4), jnp.float32),
        scratch_shapes=[pltpu.VMEM((2, B, C), jnp.float32)],
    )(domains, xr, tr, xr, tr, xr, tr, xr, tr)
    return (out[0, 0], (out[0, 1], out[0, 2]))


# speedup vs baseline: 1.5771x; 1.0047x over previous
import jax
import jax.numpy as jnp
from jax.experimental import pallas as pl
from jax.experimental.pallas import tpu as pltpu

EPS = 1e-07
B, C, H, W = 16, 4, 512, 512
NSTREAM = 8
STEPS = (B * C) // NSTREAM


def _dice_kernel(*refs):
    dom_ref = refs[0]
    pair_refs = refs[1:1 + 2 * NSTREAM]
    out_ref = refs[1 + 2 * NSTREAM]
    acc_ref = refs[2 + 2 * NSTREAM]
    i = pl.program_id(0)
    n = pl.num_programs(0)

    @pl.when(i == 0)
    def _init():
        acc_ref[...] = jnp.zeros_like(acc_ref)

    row = jax.lax.broadcasted_iota(jnp.int32, (B, C), 0)
    col = jax.lax.broadcasted_iota(jnp.int32, (B, C), 1)
    acc_i = jnp.zeros((B, C), jnp.float32)
    acc_c = jnp.zeros((B, C), jnp.float32)
    for q in range(NSTREAM):
        xq = pair_refs[2 * q][0]
        tq = pair_refs[2 * q + 1][0]
        slab = i + q * STEPS
        hot = (row == slab // C) & (col == slab % C)
        acc_i += jnp.where(hot, jnp.sum(xq * tq), 0.0)
        acc_c += jnp.where(hot, jnp.sum(xq + tq), 0.0)
    acc_ref[0] += acc_i
    acc_ref[1] += acc_c

    @pl.when(i == n - 1)
    def _epilogue():
        inter = acc_ref[0]
        card = acc_ref[1]
        d0 = dom_ref[:, 0:1]
        d1 = dom_ref[:, 1:2]
        w1 = (d1 > d0).astype(jnp.float32)
        w0 = 1.0 - w1
        i0 = jnp.sum(inter * w0, axis=0, keepdims=True)
        c0 = jnp.sum(card * w0, axis=0, keepdims=True)
        i1 = jnp.sum(inter * w1, axis=0, keepdims=True)
        c1 = jnp.sum(card * w1, axis=0, keepdims=True)
        loss0 = 1.0 - jnp.mean(2.0 * i0 / (c0 + EPS))
        loss1 = 1.0 - jnp.mean(2.0 * i1 / (c1 + EPS))
        lane = jax.lax.broadcasted_iota(jnp.int32, (1, 4), 1)
        out_ref[...] = jnp.where(
            lane == 0, loss0 + loss1, jnp.where(lane == 1, loss0, loss1)
        )


def kernel(x, label_true, domains):
    xr = x.reshape(B * C, H, W)
    tr = label_true.reshape(B * C, H, W)
    specs = [pl.BlockSpec((B, 2), lambda i: (0, 0))]
    operands = [domains]
    for q in range(NSTREAM):
        specs.append(pl.BlockSpec((1, H, W), lambda i, q=q: (i + q * STEPS, 0, 0)))
        specs.append(pl.BlockSpec((1, H, W), lambda i, q=q: (i + q * STEPS, 0, 0)))
        operands.append(xr)
        operands.append(tr)
    out = pl.pallas_call(
        _dice_kernel,
        grid=(STEPS,),
        in_specs=specs,
        out_specs=pl.BlockSpec((1, 4), lambda i: (0, 0)),
        out_shape=jax.ShapeDtypeStruct((1, 4), jnp.float32),
        scratch_shapes=[pltpu.VMEM((2, B, C), jnp.float32)],
    )(*operands)
    return (out[0, 0], (out[0, 1], out[0, 2]))
